# baseline (speedup 1.0000x reference)
import functools

import jax
import jax.numpy as jnp
from jax import lax
from jax.experimental import pallas as pl
from jax.experimental.pallas import tpu as pltpu
from jax.experimental.pallas import tpu_sc as plsc

_N = 3200
_E = 102400
_M = 8
_NPOS = 1024
_P = 8
_SIZES = (1024, 896, 768, 512)
_HID = 64
_OUT = 32

_ROWS = 32
_NCHUNK = _N // _ROWS
_EW = 2048
_NW = 32


def _adj_body(dst_hbm, src_hbm, zero_hbm, a_hbm, a_chunk, dstw, srcw):
    wid = lax.axis_index("s") * 2 + lax.axis_index("c")
    npass = (_NCHUNK + _NW - 1) // _NW

    def chunk_body(ci, carry):
        c = ci * _NW + wid

        @pl.when(c < _NCHUNK)
        def _():
            lo = c * _ROWS
            pltpu.sync_copy(zero_hbm, a_chunk)

            def wbody(w, carry2):
                pltpu.sync_copy(dst_hbm.at[pl.ds(w * _EW, _EW)], dstw)
                pltpu.sync_copy(src_hbm.at[pl.ds(w * _EW, _EW)], srcw)

                def vbody(v, carry3):
                    d = dstw[pl.ds(v * 16, 16)]
                    s = srcw[pl.ds(v * 16, 16)]
                    m = (d >= lo) & (d < lo + _ROWS)
                    r = jnp.where(m, d - lo, 0)
                    sc = jnp.where(m, s, 0)
                    key = r * _N + sc
                    cnt, last = plsc.scan_count(key, mask=m)
                    plsc.addupdate_scatter(
                        a_chunk, [r, sc], cnt.astype(jnp.float32), mask=last)
                    return carry3

                return lax.fori_loop(0, _EW // 16, vbody, carry2)

            lax.fori_loop(0, _E // _EW, wbody, 0)
            pltpu.sync_copy(a_chunk, a_hbm.at[pl.ds(lo, _ROWS), :])

        return carry

    lax.fori_loop(0, npass, chunk_body, 0)


@functools.cache
def _get_adj_kernel():
    return pl.kernel(
        _adj_body,
        out_type=jax.ShapeDtypeStruct((_N, _N), jnp.float32),
        mesh=plsc.VectorSubcoreMesh(core_axis_name="c", subcore_axis_name="s"),
        scratch_types=[
            pltpu.VMEM((_ROWS, _N), jnp.float32),
            pltpu.VMEM((_EW,), jnp.int32),
            pltpu.VMEM((_EW,), jnp.int32),
        ],
        compiler_params=pltpu.CompilerParams(needs_layout_passes=False),
    )


def _gine_body(a_ref, w0, w1, w2, w3,
               k10, b10, k20, b20,
               k11, b11, k21, b21,
               k12, b12, k22, b22,
               out_ref):
    j = pl.program_id(0)
    a = a_ref[...]

    xs = []
    for ref, n in zip((w0, w1, w2, w3), _SIZES):
        g = (j < n // _P).astype(jnp.float32)
        xs.append(jnp.squeeze(ref[...], axis=0) * g)
    x = jnp.concatenate(xs, axis=0)

    for li, (k1, b1, k2, b2) in enumerate(((k10, b10, k20, b20),
                                           (k11, b11, k21, b21),
                                           (k12, b12, k22, b22))):
        r = jnp.maximum(x, 0.0).astype(jnp.bfloat16)
        agg = jnp.dot(a, r, preferred_element_type=jnp.float32)
        y = (x + agg).astype(jnp.bfloat16)
        h = jnp.dot(y, k1[...], preferred_element_type=jnp.float32)
        h = jnp.maximum(h + b1[...][0:1, :], 0.0).astype(jnp.bfloat16)
        x = jnp.dot(h, k2[...], preferred_element_type=jnp.float32)
        x = x + b2[...][0:1, :]

    contrib = x

    @pl.when(j == 0)
    def _():
        out_ref[...] = contrib

    @pl.when(j > 0)
    def _():
        out_ref[...] += contrib


def _const_spec(shape):
    return pl.BlockSpec(shape, lambda j: tuple(0 for _ in shape))


def _gine_call(A, Ws, kbs):
    in_specs = [_const_spec((_N, _N))]
    for n in _SIZES:
        last = n // _P - 1
        in_specs.append(pl.BlockSpec(
            (1, n, _P * _M), functools.partial(
                lambda j, l: (jnp.minimum(j, l), 0, 0), l=last)))
    for (k1, b1, k2, b2) in kbs:
        in_specs.append(_const_spec(k1.shape))
        in_specs.append(_const_spec(b1.shape))
        in_specs.append(_const_spec(k2.shape))
        in_specs.append(_const_spec(b2.shape))

    flat = []
    for (k1, b1, k2, b2) in kbs:
        flat += [k1, b1, k2, b2]

    return pl.pallas_call(
        _gine_body,
        grid=(_NPOS // _P,),
        in_specs=in_specs,
        out_specs=pl.BlockSpec((_N, _OUT), lambda j: (0, 0)),
        out_shape=jax.ShapeDtypeStruct((_N, _OUT), jnp.float32),
        compiler_params=pltpu.CompilerParams(
            dimension_semantics=("arbitrary",),
            vmem_limit_bytes=120 * 1024 * 1024,
        ),
    )(A, *Ws, *flat)


def _prep_layer(w1, b1, w2, b2, last=False):
    eye_p = jnp.eye(_P, dtype=jnp.float32)
    k1 = jnp.kron(eye_p, w1)
    b1t = jnp.tile(jnp.tile(b1, _P)[None, :], (8, 1))
    if last:
        k2 = jnp.kron(jnp.ones((_P, 1), jnp.float32), w2)
        b2t = jnp.tile((_P * b2)[None, :], (8, 1))
    else:
        k2 = jnp.kron(eye_p, w2)
        b2t = jnp.tile(jnp.tile(b2, _P)[None, :], (8, 1))
    return (k1.astype(jnp.bfloat16), b1t,
            k2.astype(jnp.bfloat16), b2t)


def kernel(W_0, W_1, W_2, W_3, edge_index,
           p0_w1, p0_b1, p0_w2, p0_b2,
           p1_w1, p1_b1, p1_w2, p1_b2,
           p2_w1, p2_b1, p2_w2, p2_b2):
    src = edge_index[0].astype(jnp.int32)
    dst = edge_index[1].astype(jnp.int32)
    A = _get_adj_kernel()(dst, src, jnp.zeros((_ROWS, _N), jnp.float32))
    A = A.astype(jnp.bfloat16)

    Ws = [W.reshape(n, n // _P, _P * _M).transpose(1, 0, 2)
          for W, n in zip((W_0, W_1, W_2, W_3), _SIZES)]

    kbs = [_prep_layer(p0_w1, p0_b1, p0_w2, p0_b2),
           _prep_layer(p1_w1, p1_b1, p1_w2, p1_b2),
           _prep_layer(p2_w1, p2_b1, p2_w2, p2_b2, last=True)]

    return _gine_call(A, Ws, kbs)


# speedup vs baseline: 1.0293x; 1.0293x over previous
import functools

import jax
import jax.numpy as jnp
from jax import lax
from jax.experimental import pallas as pl
from jax.experimental.pallas import tpu as pltpu
from jax.experimental.pallas import tpu_sc as plsc

_N = 3200
_E = 102400
_M = 8
_NPOS = 1024
_P = 8
_SIZES = (1024, 896, 768, 512)
_HID = 64
_OUT = 32

_ROWS = 32
_NCHUNK = _N // _ROWS
_EW = 2048
_NW = 32


_NWIN = _E // _EW


def _adj_body(dst_hbm, src_hbm, zero_hbm, a_hbm, a_chunk, dstw, srcw,
              sd0, ss0, sd1, ss1):
    wid = lax.axis_index("s") * 2 + lax.axis_index("c")
    npass = (_NCHUNK + _NW - 1) // _NW
    sems = ((sd0, ss0), (sd1, ss1))

    def issue(w, b):
        pltpu.async_copy(dst_hbm.at[pl.ds(w * _EW, _EW)], dstw.at[b],
                         sems[b][0])
        pltpu.async_copy(src_hbm.at[pl.ds(w * _EW, _EW)], srcw.at[b],
                         sems[b][1])

    def wait(w, b):
        pltpu.make_async_copy(dst_hbm.at[pl.ds(w * _EW, _EW)], dstw.at[b],
                              sems[b][0]).wait()
        pltpu.make_async_copy(src_hbm.at[pl.ds(w * _EW, _EW)], srcw.at[b],
                              sems[b][1]).wait()

    def chunk_body(ci, carry):
        c = ci * _NW + wid

        @pl.when(c < _NCHUNK)
        def _():
            lo = c * _ROWS
            pltpu.sync_copy(zero_hbm, a_chunk)

            def process(b):
                def vbody(v, carry3):
                    d = dstw[b, pl.ds(v * 16, 16)]
                    s = srcw[b, pl.ds(v * 16, 16)]
                    m = (d >= lo) & (d < lo + _ROWS)
                    r = jnp.where(m, d - lo, 0)
                    sc = jnp.where(m, s, 0)
                    key = r * _N + sc
                    cnt, last = plsc.scan_count(key, mask=m)
                    plsc.addupdate_scatter(
                        a_chunk, [r, sc], cnt.astype(jnp.float32), mask=last)
                    return carry3

                lax.fori_loop(0, _EW // 16, vbody, 0)

            issue(0, 0)

            def pair_body(p, carry2):
                w0 = 2 * p
                issue(w0 + 1, 1)
                wait(w0, 0)
                process(0)

                @pl.when(w0 + 2 < _NWIN)
                def _():
                    issue(w0 + 2, 0)

                wait(w0 + 1, 1)
                process(1)
                return carry2

            lax.fori_loop(0, _NWIN // 2, pair_body, 0)
            pltpu.sync_copy(a_chunk, a_hbm.at[pl.ds(lo, _ROWS), :])

        return carry

    lax.fori_loop(0, npass, chunk_body, 0)


@functools.cache
def _get_adj_kernel():
    return pl.kernel(
        _adj_body,
        out_type=jax.ShapeDtypeStruct((_N, _N), jnp.float32),
        mesh=plsc.VectorSubcoreMesh(core_axis_name="c", subcore_axis_name="s"),
        scratch_types=[
            pltpu.VMEM((_ROWS, _N), jnp.float32),
            pltpu.VMEM((2, _EW), jnp.int32),
            pltpu.VMEM((2, _EW), jnp.int32),
            pltpu.SemaphoreType.DMA,
            pltpu.SemaphoreType.DMA,
            pltpu.SemaphoreType.DMA,
            pltpu.SemaphoreType.DMA,
        ],
        compiler_params=pltpu.CompilerParams(needs_layout_passes=False),
    )


def _gine_body(a_ref, w0, w1, w2, w3,
               k10, b10, k20, b20,
               k11, b11, k21, b21,
               k12, b12, k22, b22,
               out_ref):
    j = pl.program_id(0)
    a = a_ref[...]

    xs = []
    for ref, n in zip((w0, w1, w2, w3), _SIZES):
        g = (j < n // _P).astype(jnp.float32)
        xs.append(jnp.squeeze(ref[...], axis=0) * g)
    x = jnp.concatenate(xs, axis=0)

    for li, (k1, b1, k2, b2) in enumerate(((k10, b10, k20, b20),
                                           (k11, b11, k21, b21),
                                           (k12, b12, k22, b22))):
        r = jnp.maximum(x, 0.0).astype(jnp.bfloat16)
        agg = jnp.dot(a, r, preferred_element_type=jnp.float32)
        y = (x + agg).astype(jnp.bfloat16)
        h = jnp.dot(y, k1[...], preferred_element_type=jnp.float32)
        h = jnp.maximum(h + b1[...][0:1, :], 0.0).astype(jnp.bfloat16)
        x = jnp.dot(h, k2[...], preferred_element_type=jnp.float32)
        x = x + b2[...][0:1, :]

    contrib = x

    @pl.when(j == 0)
    def _():
        out_ref[...] = contrib

    @pl.when(j > 0)
    def _():
        out_ref[...] += contrib


def _const_spec(shape):
    return pl.BlockSpec(shape, lambda j: tuple(0 for _ in shape))


def _gine_call(A, Ws, kbs):
    in_specs = [_const_spec((_N, _N))]
    for n in _SIZES:
        last = n // _P - 1
        in_specs.append(pl.BlockSpec(
            (1, n, _P * _M), functools.partial(
                lambda j, l: (jnp.minimum(j, l), 0, 0), l=last)))
    for (k1, b1, k2, b2) in kbs:
        in_specs.append(_const_spec(k1.shape))
        in_specs.append(_const_spec(b1.shape))
        in_specs.append(_const_spec(k2.shape))
        in_specs.append(_const_spec(b2.shape))

    flat = []
    for (k1, b1, k2, b2) in kbs:
        flat += [k1, b1, k2, b2]

    return pl.pallas_call(
        _gine_body,
        grid=(_NPOS // _P,),
        in_specs=in_specs,
        out_specs=pl.BlockSpec((_N, _OUT), lambda j: (0, 0)),
        out_shape=jax.ShapeDtypeStruct((_N, _OUT), jnp.float32),
        compiler_params=pltpu.CompilerParams(
            dimension_semantics=("arbitrary",),
            vmem_limit_bytes=120 * 1024 * 1024,
        ),
    )(A, *Ws, *flat)


def _prep_layer(w1, b1, w2, b2, last=False):
    eye_p = jnp.eye(_P, dtype=jnp.float32)
    k1 = jnp.kron(eye_p, w1)
    b1t = jnp.tile(jnp.tile(b1, _P)[None, :], (8, 1))
    if last:
        k2 = jnp.kron(jnp.ones((_P, 1), jnp.float32), w2)
        b2t = jnp.tile((_P * b2)[None, :], (8, 1))
    else:
        k2 = jnp.kron(eye_p, w2)
        b2t = jnp.tile(jnp.tile(b2, _P)[None, :], (8, 1))
    return (k1.astype(jnp.bfloat16), b1t,
            k2.astype(jnp.bfloat16), b2t)


def kernel(W_0, W_1, W_2, W_3, edge_index,
           p0_w1, p0_b1, p0_w2, p0_b2,
           p1_w1, p1_b1, p1_w2, p1_b2,
           p2_w1, p2_b1, p2_w2, p2_b2):
    src = edge_index[0].astype(jnp.int32)
    dst = edge_index[1].astype(jnp.int32)
    A = _get_adj_kernel()(dst, src, jnp.zeros((_ROWS, _N), jnp.float32))
    A = A.astype(jnp.bfloat16)

    Ws = [W.reshape(n, n // _P, _P * _M).transpose(1, 0, 2)
          for W, n in zip((W_0, W_1, W_2, W_3), _SIZES)]

    kbs = [_prep_layer(p0_w1, p0_b1, p0_w2, p0_b2),
           _prep_layer(p1_w1, p1_b1, p1_w2, p1_b2),
           _prep_layer(p2_w1, p2_b1, p2_w2, p2_b2, last=True)]

    return _gine_call(A, Ws, kbs)


# speedup vs baseline: 1.0883x; 1.0574x over previous
import functools

import jax
import jax.numpy as jnp
from jax import lax
from jax.experimental import pallas as pl
from jax.experimental.pallas import tpu as pltpu
from jax.experimental.pallas import tpu_sc as plsc

_N = 3200
_E = 102400
_M = 8
_NPOS = 1024
_P = 8
_SIZES = (1024, 896, 768, 512)
_HID = 64
_OUT = 32

_ROWS = 32
_NCHUNK = _N // _ROWS
_EW = 2048
_NW = 32


_NWIN = _E // _EW


def _adj_body(dst_hbm, src_hbm, zero_hbm, a_hbm, a_chunk, dstw, srcw,
              sd0, ss0, sd1, ss1):
    wid = lax.axis_index("s") * 2 + lax.axis_index("c")
    npass = (_NCHUNK + _NW - 1) // _NW
    sems = ((sd0, ss0), (sd1, ss1))

    def issue(w, b):
        pltpu.async_copy(dst_hbm.at[pl.ds(w * _EW, _EW)], dstw.at[b],
                         sems[b][0])
        pltpu.async_copy(src_hbm.at[pl.ds(w * _EW, _EW)], srcw.at[b],
                         sems[b][1])

    def wait(w, b):
        pltpu.make_async_copy(dst_hbm.at[pl.ds(w * _EW, _EW)], dstw.at[b],
                              sems[b][0]).wait()
        pltpu.make_async_copy(src_hbm.at[pl.ds(w * _EW, _EW)], srcw.at[b],
                              sems[b][1]).wait()

    def chunk_body(ci, carry):
        c = ci * _NW + wid

        @pl.when(c < _NCHUNK)
        def _():
            lo = c * _ROWS
            pltpu.sync_copy(zero_hbm, a_chunk)

            def process(b):
                def vbody(v, carry3):
                    d = dstw[b, pl.ds(v * 16, 16)]
                    s = srcw[b, pl.ds(v * 16, 16)]
                    m = (d >= lo) & (d < lo + _ROWS)
                    r = jnp.where(m, d - lo, 0)
                    sc = jnp.where(m, s, 0)
                    key = r * _N + sc
                    cnt, last = plsc.scan_count(key, mask=m)
                    plsc.addupdate_scatter(
                        a_chunk, [r, sc], cnt.astype(jnp.float32), mask=last)
                    return carry3

                lax.fori_loop(0, _EW // 16, vbody, 0)

            issue(0, 0)

            def pair_body(p, carry2):
                w0 = 2 * p
                issue(w0 + 1, 1)
                wait(w0, 0)
                process(0)

                @pl.when(w0 + 2 < _NWIN)
                def _():
                    issue(w0 + 2, 0)

                wait(w0 + 1, 1)
                process(1)
                return carry2

            lax.fori_loop(0, _NWIN // 2, pair_body, 0)
            pltpu.sync_copy(a_chunk, a_hbm.at[pl.ds(lo, _ROWS), :])

        return carry

    lax.fori_loop(0, npass, chunk_body, 0)


@functools.cache
def _get_adj_kernel():
    return pl.kernel(
        _adj_body,
        out_type=jax.ShapeDtypeStruct((_N, _N), jnp.float32),
        mesh=plsc.VectorSubcoreMesh(core_axis_name="c", subcore_axis_name="s"),
        scratch_types=[
            pltpu.VMEM((_ROWS, _N), jnp.float32),
            pltpu.VMEM((2, _EW), jnp.int32),
            pltpu.VMEM((2, _EW), jnp.int32),
            pltpu.SemaphoreType.DMA,
            pltpu.SemaphoreType.DMA,
            pltpu.SemaphoreType.DMA,
            pltpu.SemaphoreType.DMA,
        ],
        compiler_params=pltpu.CompilerParams(needs_layout_passes=False),
    )


def _gine_body(a_ref, w0, w1, w2, w3,
               k10, b10, k20, b20,
               k11, b11, k21, b21,
               k12, b12, k22, b22,
               out_ref):
    j = pl.program_id(0)
    a = a_ref[...]

    xs = []
    for ref, n in zip((w0, w1, w2, w3), _SIZES):
        g = (j < n // _P).astype(jnp.float32)
        xs.append(jnp.squeeze(ref[...], axis=0) * g)
    x = jnp.concatenate(xs, axis=0)

    def _split_dot(v, k):
        half = v.shape[1] // 2
        lo = jnp.dot(v[:, :half], k, preferred_element_type=jnp.float32)
        hi = jnp.dot(v[:, half:], k, preferred_element_type=jnp.float32)
        return lo, hi

    for li, (k1, b1, k2, b2) in enumerate(((k10, b10, k20, b20),
                                           (k11, b11, k21, b21),
                                           (k12, b12, k22, b22))):
        r = jnp.maximum(x, 0.0).astype(jnp.bfloat16)
        agg = jnp.dot(a, r, preferred_element_type=jnp.float32)
        y = (x + agg).astype(jnp.bfloat16)
        hlo, hhi = _split_dot(y, k1[...])
        h = jnp.concatenate([hlo, hhi], axis=1)
        h = jnp.maximum(h + b1[...][0:1, :], 0.0).astype(jnp.bfloat16)
        xlo, xhi = _split_dot(h, k2[...])
        if li == 2:
            x = xlo + xhi + b2[...][0:1, :]
        else:
            x = jnp.concatenate([xlo, xhi], axis=1) + b2[...][0:1, :]

    contrib = x

    @pl.when(j == 0)
    def _():
        out_ref[...] = contrib

    @pl.when(j > 0)
    def _():
        out_ref[...] += contrib


def _const_spec(shape):
    return pl.BlockSpec(shape, lambda j: tuple(0 for _ in shape))


def _gine_call(A, Ws, kbs):
    in_specs = [_const_spec((_N, _N))]
    for n in _SIZES:
        last = n // _P - 1
        in_specs.append(pl.BlockSpec(
            (1, n, _P * _M), functools.partial(
                lambda j, l: (jnp.minimum(j, l), 0, 0), l=last)))
    for (k1, b1, k2, b2) in kbs:
        in_specs.append(_const_spec(k1.shape))
        in_specs.append(_const_spec(b1.shape))
        in_specs.append(_const_spec(k2.shape))
        in_specs.append(_const_spec(b2.shape))

    flat = []
    for (k1, b1, k2, b2) in kbs:
        flat += [k1, b1, k2, b2]

    return pl.pallas_call(
        _gine_body,
        grid=(_NPOS // _P,),
        in_specs=in_specs,
        out_specs=pl.BlockSpec((_N, _OUT), lambda j: (0, 0)),
        out_shape=jax.ShapeDtypeStruct((_N, _OUT), jnp.float32),
        compiler_params=pltpu.CompilerParams(
            dimension_semantics=("arbitrary",),
            vmem_limit_bytes=120 * 1024 * 1024,
        ),
    )(A, *Ws, *flat)


def _prep_layer(w1, b1, w2, b2, last=False):
    eye_h = jnp.eye(_P // 2, dtype=jnp.float32)
    k1 = jnp.kron(eye_h, w1)
    b1t = jnp.tile(jnp.tile(b1, _P)[None, :], (8, 1))
    if last:
        k2 = jnp.kron(jnp.ones((_P // 2, 1), jnp.float32), w2)
        b2t = jnp.tile((_P * b2)[None, :], (8, 1))
    else:
        k2 = jnp.kron(eye_h, w2)
        b2t = jnp.tile(jnp.tile(b2, _P)[None, :], (8, 1))
    return (k1.astype(jnp.bfloat16), b1t,
            k2.astype(jnp.bfloat16), b2t)


def kernel(W_0, W_1, W_2, W_3, edge_index,
           p0_w1, p0_b1, p0_w2, p0_b2,
           p1_w1, p1_b1, p1_w2, p1_b2,
           p2_w1, p2_b1, p2_w2, p2_b2):
    src = edge_index[0].astype(jnp.int32)
    dst = edge_index[1].astype(jnp.int32)
    A = _get_adj_kernel()(dst, src, jnp.zeros((_ROWS, _N), jnp.float32))
    A = A.astype(jnp.bfloat16)

    Ws = [W.reshape(n, n // _P, _P * _M).transpose(1, 0, 2)
          for W, n in zip((W_0, W_1, W_2, W_3), _SIZES)]

    kbs = [_prep_layer(p0_w1, p0_b1, p0_w2, p0_b2),
           _prep_layer(p1_w1, p1_b1, p1_w2, p1_b2),
           _prep_layer(p2_w1, p2_b1, p2_w2, p2_b2, last=True)]

    return _gine_call(A, Ws, kbs)


# speedup vs baseline: 1.0886x; 1.0002x over previous
import functools

import jax
import jax.numpy as jnp
from jax import lax
from jax.experimental import pallas as pl
from jax.experimental.pallas import tpu as pltpu
from jax.experimental.pallas import tpu_sc as plsc

_N = 3200
_E = 102400
_M = 8
_NPOS = 1024
_P = 8
_G = 4
_SIZES = (1024, 896, 768, 512)
_HID = 64
_OUT = 32

_ROWS = 32
_NCHUNK = _N // _ROWS
_EW = 2048
_NW = 32


_NWIN = _E // _EW


def _adj_body(dst_hbm, src_hbm, zero_hbm, a_hbm, a_chunk, dstw, srcw,
              sd0, ss0, sd1, ss1):
    wid = lax.axis_index("s") * 2 + lax.axis_index("c")
    npass = (_NCHUNK + _NW - 1) // _NW
    sems = ((sd0, ss0), (sd1, ss1))

    def issue(w, b):
        pltpu.async_copy(dst_hbm.at[pl.ds(w * _EW, _EW)], dstw.at[b],
                         sems[b][0])
        pltpu.async_copy(src_hbm.at[pl.ds(w * _EW, _EW)], srcw.at[b],
                         sems[b][1])

    def wait(w, b):
        pltpu.make_async_copy(dst_hbm.at[pl.ds(w * _EW, _EW)], dstw.at[b],
                              sems[b][0]).wait()
        pltpu.make_async_copy(src_hbm.at[pl.ds(w * _EW, _EW)], srcw.at[b],
                              sems[b][1]).wait()

    def chunk_body(ci, carry):
        c = ci * _NW + wid

        @pl.when(c < _NCHUNK)
        def _():
            lo = c * _ROWS
            pltpu.sync_copy(zero_hbm, a_chunk)

            def process(b):
                def vbody(v, carry3):
                    d = dstw[b, pl.ds(v * 16, 16)]
                    s = srcw[b, pl.ds(v * 16, 16)]
                    m = (d >= lo) & (d < lo + _ROWS)
                    r = jnp.where(m, d - lo, 0)
                    sc = jnp.where(m, s, 0)
                    key = r * _N + sc
                    cnt, last = plsc.scan_count(key, mask=m)
                    plsc.addupdate_scatter(
                        a_chunk, [r, sc], cnt.astype(jnp.float32), mask=last)
                    return carry3

                lax.fori_loop(0, _EW // 16, vbody, 0)

            issue(0, 0)

            def pair_body(p, carry2):
                w0 = 2 * p
                issue(w0 + 1, 1)
                wait(w0, 0)
                process(0)

                @pl.when(w0 + 2 < _NWIN)
                def _():
                    issue(w0 + 2, 0)

                wait(w0 + 1, 1)
                process(1)
                return carry2

            lax.fori_loop(0, _NWIN // 2, pair_body, 0)
            pltpu.sync_copy(a_chunk, a_hbm.at[pl.ds(lo, _ROWS), :])

        return carry

    lax.fori_loop(0, npass, chunk_body, 0)


@functools.cache
def _get_adj_kernel():
    return pl.kernel(
        _adj_body,
        out_type=jax.ShapeDtypeStruct((_N, _N), jnp.float32),
        mesh=plsc.VectorSubcoreMesh(core_axis_name="c", subcore_axis_name="s"),
        scratch_types=[
            pltpu.VMEM((_ROWS, _N), jnp.float32),
            pltpu.VMEM((2, _EW), jnp.int32),
            pltpu.VMEM((2, _EW), jnp.int32),
            pltpu.SemaphoreType.DMA,
            pltpu.SemaphoreType.DMA,
            pltpu.SemaphoreType.DMA,
            pltpu.SemaphoreType.DMA,
        ],
        compiler_params=pltpu.CompilerParams(needs_layout_passes=False),
    )


def _gine_body(a_ref, w0, w1, w2, w3,
               k10, b10, k20, b20,
               k11, b11, k21, b21,
               k12, b12, k22, b22,
               out_ref):
    j = pl.program_id(0)
    a = a_ref[...]

    xs = []
    for ref, n in zip((w0, w1, w2, w3), _SIZES):
        g = (j < n // _P).astype(jnp.float32)
        xs.append(jnp.squeeze(ref[...], axis=0) * g)
    x = jnp.concatenate(xs, axis=0)

    def _split_dot(v, k):
        piece = v.shape[1] // _G
        return [jnp.dot(v[:, g * piece:(g + 1) * piece], k,
                        preferred_element_type=jnp.float32)
                for g in range(_G)]

    for li, (k1, b1, k2, b2) in enumerate(((k10, b10, k20, b20),
                                           (k11, b11, k21, b21),
                                           (k12, b12, k22, b22))):
        r = jnp.maximum(x, 0.0).astype(jnp.bfloat16)
        agg = jnp.dot(a, r, preferred_element_type=jnp.float32)
        y = (x + agg).astype(jnp.bfloat16)
        hs = _split_dot(y, k1[...])
        h = jnp.concatenate(hs, axis=1)
        h = jnp.maximum(h + b1[...][0:1, :], 0.0).astype(jnp.bfloat16)
        xs2 = _split_dot(h, k2[...])
        if li == 2:
            x = sum(xs2) + b2[...][0:1, :]
        else:
            x = jnp.concatenate(xs2, axis=1) + b2[...][0:1, :]

    contrib = x

    @pl.when(j == 0)
    def _():
        out_ref[...] = contrib

    @pl.when(j > 0)
    def _():
        out_ref[...] += contrib


def _const_spec(shape):
    return pl.BlockSpec(shape, lambda j: tuple(0 for _ in shape))


def _gine_call(A, Ws, kbs):
    in_specs = [_const_spec((_N, _N))]
    for n in _SIZES:
        last = n // _P - 1
        in_specs.append(pl.BlockSpec(
            (1, n, _P * _M), functools.partial(
                lambda j, l: (jnp.minimum(j, l), 0, 0), l=last)))
    for (k1, b1, k2, b2) in kbs:
        in_specs.append(_const_spec(k1.shape))
        in_specs.append(_const_spec(b1.shape))
        in_specs.append(_const_spec(k2.shape))
        in_specs.append(_const_spec(b2.shape))

    flat = []
    for (k1, b1, k2, b2) in kbs:
        flat += [k1, b1, k2, b2]

    return pl.pallas_call(
        _gine_body,
        grid=(_NPOS // _P,),
        in_specs=in_specs,
        out_specs=pl.BlockSpec((_N, _OUT), lambda j: (0, 0)),
        out_shape=jax.ShapeDtypeStruct((_N, _OUT), jnp.float32),
        compiler_params=pltpu.CompilerParams(
            dimension_semantics=("arbitrary",),
            vmem_limit_bytes=120 * 1024 * 1024,
        ),
    )(A, *Ws, *flat)


def _prep_layer(w1, b1, w2, b2, last=False):
    eye_h = jnp.eye(_P // _G, dtype=jnp.float32)
    k1 = jnp.kron(eye_h, w1)
    b1t = jnp.tile(jnp.tile(b1, _P)[None, :], (8, 1))
    if last:
        k2 = jnp.kron(jnp.ones((_P // _G, 1), jnp.float32), w2)
        b2t = jnp.tile((_P * b2)[None, :], (8, 1))
    else:
        k2 = jnp.kron(eye_h, w2)
        b2t = jnp.tile(jnp.tile(b2, _P)[None, :], (8, 1))
    return (k1.astype(jnp.bfloat16), b1t,
            k2.astype(jnp.bfloat16), b2t)


def kernel(W_0, W_1, W_2, W_3, edge_index,
           p0_w1, p0_b1, p0_w2, p0_b2,
           p1_w1, p1_b1, p1_w2, p1_b2,
           p2_w1, p2_b1, p2_w2, p2_b2):
    src = edge_index[0].astype(jnp.int32)
    dst = edge_index[1].astype(jnp.int32)
    A = _get_adj_kernel()(dst, src, jnp.zeros((_ROWS, _N), jnp.float32))
    A = A.astype(jnp.bfloat16)

    Ws = [W.reshape(n, n // _P, _P * _M).transpose(1, 0, 2)
          for W, n in zip((W_0, W_1, W_2, W_3), _SIZES)]

    kbs = [_prep_layer(p0_w1, p0_b1, p0_w2, p0_b2),
           _prep_layer(p1_w1, p1_b1, p1_w2, p1_b2),
           _prep_layer(p2_w1, p2_b1, p2_w2, p2_b2, last=True)]

    return _gine_call(A, Ws, kbs)


# speedup vs baseline: 1.4057x; 1.2914x over previous
import functools

import jax
import jax.numpy as jnp
from jax import lax
from jax.experimental import pallas as pl
from jax.experimental.pallas import tpu as pltpu
from jax.experimental.pallas import tpu_sc as plsc

_N = 3200
_E = 102400
_M = 8
_NPOS = 1024
_P = 16
_G = 8
_SIZES = (1024, 896, 768, 512)
_HID = 64
_OUT = 32

_ROWS = 32
_NCHUNK = _N // _ROWS
_EW = 2048
_NW = 32


_NWIN = _E // _EW


def _adj_body(dst_hbm, src_hbm, zero_hbm, a_hbm, a_chunk, dstw, srcw,
              sd0, ss0, sd1, ss1):
    wid = lax.axis_index("s") * 2 + lax.axis_index("c")
    npass = (_NCHUNK + _NW - 1) // _NW
    sems = ((sd0, ss0), (sd1, ss1))

    def issue(w, b):
        pltpu.async_copy(dst_hbm.at[pl.ds(w * _EW, _EW)], dstw.at[b],
                         sems[b][0])
        pltpu.async_copy(src_hbm.at[pl.ds(w * _EW, _EW)], srcw.at[b],
                         sems[b][1])

    def wait(w, b):
        pltpu.make_async_copy(dst_hbm.at[pl.ds(w * _EW, _EW)], dstw.at[b],
                              sems[b][0]).wait()
        pltpu.make_async_copy(src_hbm.at[pl.ds(w * _EW, _EW)], srcw.at[b],
                              sems[b][1]).wait()

    def chunk_body(ci, carry):
        c = ci * _NW + wid

        @pl.when(c < _NCHUNK)
        def _():
            lo = c * _ROWS
            pltpu.sync_copy(zero_hbm, a_chunk)

            def process(b):
                def vbody(v, carry3):
                    d = dstw[b, pl.ds(v * 16, 16)]
                    s = srcw[b, pl.ds(v * 16, 16)]
                    m = (d >= lo) & (d < lo + _ROWS)
                    r = jnp.where(m, d - lo, 0)
                    sc = jnp.where(m, s, 0)
                    key = r * _N + sc
                    cnt, last = plsc.scan_count(key, mask=m)
                    plsc.addupdate_scatter(
                        a_chunk, [r, sc], cnt.astype(jnp.float32), mask=last)
                    return carry3

                lax.fori_loop(0, _EW // 16, vbody, 0)

            issue(0, 0)

            def pair_body(p, carry2):
                w0 = 2 * p
                issue(w0 + 1, 1)
                wait(w0, 0)
                process(0)

                @pl.when(w0 + 2 < _NWIN)
                def _():
                    issue(w0 + 2, 0)

                wait(w0 + 1, 1)
                process(1)
                return carry2

            lax.fori_loop(0, _NWIN // 2, pair_body, 0)
            pltpu.sync_copy(a_chunk, a_hbm.at[pl.ds(lo, _ROWS), :])

        return carry

    lax.fori_loop(0, npass, chunk_body, 0)


@functools.cache
def _get_adj_kernel():
    return pl.kernel(
        _adj_body,
        out_type=jax.ShapeDtypeStruct((_N, _N), jnp.float32),
        mesh=plsc.VectorSubcoreMesh(core_axis_name="c", subcore_axis_name="s"),
        scratch_types=[
            pltpu.VMEM((_ROWS, _N), jnp.float32),
            pltpu.VMEM((2, _EW), jnp.int32),
            pltpu.VMEM((2, _EW), jnp.int32),
            pltpu.SemaphoreType.DMA,
            pltpu.SemaphoreType.DMA,
            pltpu.SemaphoreType.DMA,
            pltpu.SemaphoreType.DMA,
        ],
        compiler_params=pltpu.CompilerParams(needs_layout_passes=False),
    )


def _gine_body(a_ref, w0, w1, w2, w3,
               k10, b10, k20, b20,
               k11, b11, k21, b21,
               k12, b12, k22, b22,
               out_ref):
    j = pl.program_id(0)
    a = a_ref[...]

    xs = []
    for ref, n in zip((w0, w1, w2, w3), _SIZES):
        g = (j < n // _P).astype(jnp.float32)
        xs.append(jnp.squeeze(ref[...], axis=0) * g)
    x = jnp.concatenate(xs, axis=0)

    def _split_dot(v, k):
        piece = v.shape[1] // _G
        return [jnp.dot(v[:, g * piece:(g + 1) * piece], k,
                        preferred_element_type=jnp.float32)
                for g in range(_G)]

    for li, (k1, b1, k2, b2) in enumerate(((k10, b10, k20, b20),
                                           (k11, b11, k21, b21),
                                           (k12, b12, k22, b22))):
        r = jnp.maximum(x, 0.0).astype(jnp.bfloat16)
        agg = jnp.dot(a, r, preferred_element_type=jnp.float32)
        y = (x + agg).astype(jnp.bfloat16)
        hs = _split_dot(y, k1[...])
        h = jnp.concatenate(hs, axis=1)
        h = jnp.maximum(h + b1[...][0:1, :], 0.0).astype(jnp.bfloat16)
        xs2 = _split_dot(h, k2[...])
        if li == 2:
            x = sum(xs2) + b2[...][0:1, :]
        else:
            x = jnp.concatenate(xs2, axis=1) + b2[...][0:1, :]

    contrib = x

    @pl.when(j == 0)
    def _():
        out_ref[...] = contrib

    @pl.when(j > 0)
    def _():
        out_ref[...] += contrib


def _const_spec(shape):
    return pl.BlockSpec(shape, lambda j: tuple(0 for _ in shape))


def _gine_call(A, Ws, kbs):
    in_specs = [_const_spec((_N, _N))]
    for n in _SIZES:
        last = n // _P - 1
        in_specs.append(pl.BlockSpec(
            (1, n, _P * _M), functools.partial(
                lambda j, l: (jnp.minimum(j, l), 0, 0), l=last)))
    for (k1, b1, k2, b2) in kbs:
        in_specs.append(_const_spec(k1.shape))
        in_specs.append(_const_spec(b1.shape))
        in_specs.append(_const_spec(k2.shape))
        in_specs.append(_const_spec(b2.shape))

    flat = []
    for (k1, b1, k2, b2) in kbs:
        flat += [k1, b1, k2, b2]

    return pl.pallas_call(
        _gine_body,
        grid=(_NPOS // _P,),
        in_specs=in_specs,
        out_specs=pl.BlockSpec((_N, _OUT), lambda j: (0, 0)),
        out_shape=jax.ShapeDtypeStruct((_N, _OUT), jnp.float32),
        compiler_params=pltpu.CompilerParams(
            dimension_semantics=("arbitrary",),
            vmem_limit_bytes=120 * 1024 * 1024,
        ),
    )(A, *Ws, *flat)


def _prep_layer(w1, b1, w2, b2, last=False):
    eye_h = jnp.eye(_P // _G, dtype=jnp.float32)
    k1 = jnp.kron(eye_h, w1)
    b1t = jnp.tile(jnp.tile(b1, _P)[None, :], (8, 1))
    if last:
        k2 = jnp.kron(jnp.ones((_P // _G, 1), jnp.float32), w2)
        b2t = jnp.tile((_P * b2)[None, :], (8, 1))
    else:
        k2 = jnp.kron(eye_h, w2)
        b2t = jnp.tile(jnp.tile(b2, _P)[None, :], (8, 1))
    return (k1.astype(jnp.bfloat16), b1t,
            k2.astype(jnp.bfloat16), b2t)


def kernel(W_0, W_1, W_2, W_3, edge_index,
           p0_w1, p0_b1, p0_w2, p0_b2,
           p1_w1, p1_b1, p1_w2, p1_b2,
           p2_w1, p2_b1, p2_w2, p2_b2):
    src = edge_index[0].astype(jnp.int32)
    dst = edge_index[1].astype(jnp.int32)
    A = _get_adj_kernel()(dst, src, jnp.zeros((_ROWS, _N), jnp.float32))
    A = A.astype(jnp.bfloat16)

    Ws = [W.reshape(n, n // _P, _P * _M).transpose(1, 0, 2)
          for W, n in zip((W_0, W_1, W_2, W_3), _SIZES)]

    kbs = [_prep_layer(p0_w1, p0_b1, p0_w2, p0_b2),
           _prep_layer(p1_w1, p1_b1, p1_w2, p1_b2),
           _prep_layer(p2_w1, p2_b1, p2_w2, p2_b2, last=True)]

    return _gine_call(A, Ws, kbs)
